# baseline (speedup 1.0000x reference)
import jax
import jax.numpy as jnp
from jax.experimental import pallas as pl
from jax.experimental.pallas import tpu as pltpu

_B, _H, _W, _J = 32, 512, 512, 17
_IMGS = 8
_R = _IMGS * _H
_NCH = _B // _IMGS
_CH = 32
_ICH = 8
_NEG_LOG2E = -1.4426950408889634


def _fold_lanes(t):
    return (t[:, 0:128] + t[:, 128:256]) + (t[:, 256:384] + t[:, 384:512])


def _fold(t):
    return _fold_lanes((t[0:8] + t[8:16]) + (t[16:24] + t[24:32]))


def _body(xs_ref, ys_ref, m_hbm, g_hbm, out_ref, bm, bg, sems):
    def copy_in(k, slot):
        pltpu.make_async_copy(
            m_hbm.at[pl.ds(k * _R, _R), :], bm.at[slot], sems.at[slot, 0]
        ).start()
        pltpu.make_async_copy(
            g_hbm.at[pl.ds(k * _R, _R), :], bg.at[slot], sems.at[slot, 1]
        ).start()

    def wait_in(k, slot):
        pltpu.make_async_copy(
            m_hbm.at[pl.ds(k * _R, _R), :], bm.at[slot], sems.at[slot, 0]
        ).wait()
        pltpu.make_async_copy(
            g_hbm.at[pl.ds(k * _R, _R), :], bg.at[slot], sems.at[slot, 1]
        ).wait()

    copy_in(0, 0)

    cols = jax.lax.broadcasted_iota(jnp.int32, (_ICH, _W), 1)
    zero_acc = jnp.zeros((8, 128), jnp.float32)

    def step(k, carry):
        a_pos, a_cnt, a_all, a_ins, area = carry
        slot = jax.lax.rem(k, 2)

        @pl.when(k + 1 < _NCH)
        def _prefetch():
            copy_in(k + 1, jax.lax.rem(k + 1, 2))

        wait_in(k, slot)

        def chunk(c, acc3):
            c_pos, c_cnt, c_all = acc3
            x = bm[slot, pl.ds(c * _CH, _CH), :]
            g = bg[slot, pl.ds(c * _CH, _CH), :]
            y = jnp.exp2(jnp.abs(x) * jnp.float32(_NEG_LOG2E))
            l_term = jnp.log(1.0 + y)
            n = jnp.maximum(x, 0.0) + l_term
            p = n - x
            pos = g > 0.0
            c_pos = c_pos + _fold(jnp.where(pos, p, 0.0))
            c_cnt = c_cnt + _fold(jnp.where(pos, 1.0, 0.0))
            c_all = c_all + _fold(n)
            return c_pos, c_cnt, c_all

        a_pos, a_cnt, a_all = jax.lax.fori_loop(
            0, _R // _CH, chunk, (a_pos, a_cnt, a_all), unroll=8)

        for i in range(_IMGS):
            b = k * _IMGS + i
            xrow = xs_ref[pl.ds(b, 1), :]
            yrow = ys_ref[pl.ds(b, 1), :]
            x_min = jnp.maximum(jnp.min(xrow).astype(jnp.int32) - 10, 0)
            x_max = jnp.minimum(jnp.max(xrow).astype(jnp.int32) + 10, _W)
            y_min = jnp.maximum(jnp.min(yrow).astype(jnp.int32) - 10, 0)
            y_max = jnp.minimum(jnp.max(yrow).astype(jnp.int32) + 10, _H)
            y_len = jnp.maximum(y_max - y_min, 0)
            x_len = jnp.maximum(x_max - x_min, 0)

            col_in = (cols - x_min).astype(jnp.uint32) < x_len.astype(
                jnp.uint32)
            row0 = i * _H
            base = row0 + y_min
            lo = row0 // _ICH + y_min // _ICH
            hi = jnp.where(
                y_len > 0, row0 // _ICH + (y_max + _ICH - 1) // _ICH, lo)

            def ins_chunk(j, a, base=base, y_len=y_len, col_in=col_in,
                          slot=slot):
                xx = bm[slot, pl.ds(j * _ICH, _ICH), :]
                yy = jnp.exp2(jnp.abs(xx) * jnp.float32(_NEG_LOG2E))
                neg_val = jnp.maximum(xx, 0.0) + jnp.log(1.0 + yy)
                rows = j * _ICH + jax.lax.broadcasted_iota(
                    jnp.int32, (_ICH, _W), 0)
                row_in = (rows - base).astype(jnp.uint32) < y_len.astype(
                    jnp.uint32)
                return a + _fold_lanes(
                    jnp.where(row_in & col_in, neg_val, 0.0))

            a_ins = jax.lax.fori_loop(lo, hi, ins_chunk, a_ins)
            area = area + (y_len * x_len).astype(jnp.float32)
        return a_pos, a_cnt, a_all, a_ins, area

    a_pos, a_cnt, a_all, a_ins, area = jax.lax.fori_loop(
        0, _NCH, step, (zero_acc, zero_acc, zero_acc, zero_acc,
                        jnp.float32(0.0)))

    out_ref[0] = jnp.sum(a_pos)
    out_ref[1] = jnp.sum(a_cnt)
    out_ref[2] = jnp.sum(a_all) - jnp.sum(a_ins)
    out_ref[3] = area


def kernel(skls, masks, gt_masks):
    s = jax.lax.stop_gradient(skls)
    xs = s[:, :, 0]
    ys = s[:, :, 1]
    m2d = masks.reshape(_B * _H, _W)
    g2d = gt_masks.reshape(_B * _H, _W)

    acc = pl.pallas_call(
        _body,
        in_specs=[
            pl.BlockSpec(memory_space=pltpu.VMEM),
            pl.BlockSpec(memory_space=pltpu.VMEM),
            pl.BlockSpec(memory_space=pl.ANY),
            pl.BlockSpec(memory_space=pl.ANY),
        ],
        out_specs=pl.BlockSpec(memory_space=pltpu.SMEM),
        out_shape=jax.ShapeDtypeStruct((4,), jnp.float32),
        scratch_shapes=[
            pltpu.VMEM((2, _R, _W), jnp.float32),
            pltpu.VMEM((2, _R, _W), jnp.float32),
            pltpu.SemaphoreType.DMA((2, 2)),
        ],
    )(xs, ys, m2d, g2d)

    pos_loss = acc[0] / jnp.maximum(acc[1], 1.0)
    neg_count = float(_B * _H * _W) - acc[3]
    neg_loss = acc[2] / jnp.maximum(neg_count, 1.0)
    return pos_loss + neg_loss


# speedup vs baseline: 1.0017x; 1.0017x over previous
import jax
import jax.numpy as jnp
from jax.experimental import pallas as pl
from jax.experimental.pallas import tpu as pltpu

_B, _H, _W, _J = 32, 512, 512, 17
_IMGS = 4
_R = _IMGS * _H
_NCH = _B // _IMGS
_CH = 64
_ICH = 8
_NEG_LOG2E = -1.4426950408889634


def _fold_lanes(t):
    return (t[:, 0:128] + t[:, 128:256]) + (t[:, 256:384] + t[:, 384:512])


def _fold(t):
    t = (t[0:16] + t[16:32]) + (t[32:48] + t[48:64])
    return _fold_lanes(t[0:8] + t[8:16])


def _body(xs_ref, ys_ref, m_hbm, g_hbm, out_ref, bm, bg, sems):
    def copy_in(k, slot):
        pltpu.make_async_copy(
            m_hbm.at[pl.ds(k * _R, _R), :], bm.at[slot], sems.at[slot, 0]
        ).start()
        pltpu.make_async_copy(
            g_hbm.at[pl.ds(k * _R, _R), :], bg.at[slot], sems.at[slot, 1]
        ).start()

    def wait_in(k, slot):
        pltpu.make_async_copy(
            m_hbm.at[pl.ds(k * _R, _R), :], bm.at[slot], sems.at[slot, 0]
        ).wait()
        pltpu.make_async_copy(
            g_hbm.at[pl.ds(k * _R, _R), :], bg.at[slot], sems.at[slot, 1]
        ).wait()

    copy_in(0, 0)

    cols = jax.lax.broadcasted_iota(jnp.int32, (_ICH, _W), 1)
    zero_acc = jnp.zeros((8, 128), jnp.float32)

    def step(k, carry):
        a_pos, a_cnt, a_all, a_ins, area = carry
        slot = jax.lax.rem(k, 2)

        @pl.when(k + 1 < _NCH)
        def _prefetch():
            copy_in(k + 1, jax.lax.rem(k + 1, 2))

        wait_in(k, slot)

        def chunk(c, acc3):
            c_pos, c_cnt, c_all = acc3
            x = bm[slot, pl.ds(c * _CH, _CH), :]
            g = bg[slot, pl.ds(c * _CH, _CH), :]
            y = jnp.exp2(jnp.abs(x) * jnp.float32(_NEG_LOG2E))
            l_term = jnp.log(1.0 + y)
            n = jnp.maximum(x, 0.0) + l_term
            p = n - x
            pos = g > 0.0
            c_pos = c_pos + _fold(jnp.where(pos, p, 0.0))
            c_cnt = c_cnt + _fold(jnp.where(pos, 1.0, 0.0))
            c_all = c_all + _fold(n)
            return c_pos, c_cnt, c_all

        a_pos, a_cnt, a_all = jax.lax.fori_loop(
            0, _R // _CH, chunk, (a_pos, a_cnt, a_all), unroll=4)

        for i in range(_IMGS):
            b = k * _IMGS + i
            xrow = xs_ref[pl.ds(b, 1), :]
            yrow = ys_ref[pl.ds(b, 1), :]
            x_min = jnp.maximum(jnp.min(xrow).astype(jnp.int32) - 10, 0)
            x_max = jnp.minimum(jnp.max(xrow).astype(jnp.int32) + 10, _W)
            y_min = jnp.maximum(jnp.min(yrow).astype(jnp.int32) - 10, 0)
            y_max = jnp.minimum(jnp.max(yrow).astype(jnp.int32) + 10, _H)
            y_len = jnp.maximum(y_max - y_min, 0)
            x_len = jnp.maximum(x_max - x_min, 0)

            col_in = (cols - x_min).astype(jnp.uint32) < x_len.astype(
                jnp.uint32)
            row0 = i * _H
            base = row0 + y_min
            lo = row0 // _ICH + y_min // _ICH
            hi = jnp.where(
                y_len > 0, row0 // _ICH + (y_max + _ICH - 1) // _ICH, lo)

            def ins_chunk(j, a, base=base, y_len=y_len, col_in=col_in,
                          slot=slot):
                xx = bm[slot, pl.ds(j * _ICH, _ICH), :]
                yy = jnp.exp2(jnp.abs(xx) * jnp.float32(_NEG_LOG2E))
                neg_val = jnp.maximum(xx, 0.0) + jnp.log(1.0 + yy)
                rows = j * _ICH + jax.lax.broadcasted_iota(
                    jnp.int32, (_ICH, _W), 0)
                row_in = (rows - base).astype(jnp.uint32) < y_len.astype(
                    jnp.uint32)
                return a + _fold_lanes(
                    jnp.where(row_in & col_in, neg_val, 0.0))

            a_ins = jax.lax.fori_loop(lo, hi, ins_chunk, a_ins)
            area = area + (y_len * x_len).astype(jnp.float32)
        return a_pos, a_cnt, a_all, a_ins, area

    a_pos, a_cnt, a_all, a_ins, area = jax.lax.fori_loop(
        0, _NCH, step, (zero_acc, zero_acc, zero_acc, zero_acc,
                        jnp.float32(0.0)))

    out_ref[0] = jnp.sum(a_pos)
    out_ref[1] = jnp.sum(a_cnt)
    out_ref[2] = jnp.sum(a_all) - jnp.sum(a_ins)
    out_ref[3] = area


def kernel(skls, masks, gt_masks):
    s = jax.lax.stop_gradient(skls)
    xs = s[:, :, 0]
    ys = s[:, :, 1]
    m2d = masks.reshape(_B * _H, _W)
    g2d = gt_masks.reshape(_B * _H, _W)

    acc = pl.pallas_call(
        _body,
        in_specs=[
            pl.BlockSpec(memory_space=pltpu.VMEM),
            pl.BlockSpec(memory_space=pltpu.VMEM),
            pl.BlockSpec(memory_space=pl.ANY),
            pl.BlockSpec(memory_space=pl.ANY),
        ],
        out_specs=pl.BlockSpec(memory_space=pltpu.SMEM),
        out_shape=jax.ShapeDtypeStruct((4,), jnp.float32),
        scratch_shapes=[
            pltpu.VMEM((2, _R, _W), jnp.float32),
            pltpu.VMEM((2, _R, _W), jnp.float32),
            pltpu.SemaphoreType.DMA((2, 2)),
        ],
    )(xs, ys, m2d, g2d)

    pos_loss = acc[0] / jnp.maximum(acc[1], 1.0)
    neg_count = float(_B * _H * _W) - acc[3]
    neg_loss = acc[2] / jnp.maximum(neg_count, 1.0)
    return pos_loss + neg_loss


# speedup vs baseline: 1.0718x; 1.0700x over previous
import jax
import jax.numpy as jnp
from jax.experimental import pallas as pl
from jax.experimental.pallas import tpu as pltpu

_B, _H, _W, _J = 32, 512, 512, 17
_IMGS = 4
_R = _IMGS * _H
_NCH = _B // _IMGS
_CH = 32
_ICH = 8
_NEG_LOG2E = -1.4426950408889634


def _fold_lanes(t):
    return (t[:, 0:128] + t[:, 128:256]) + (t[:, 256:384] + t[:, 384:512])


def _fold(t):
    return _fold_lanes((t[0:8] + t[8:16]) + (t[16:24] + t[24:32]))


def _body(xs_ref, ys_ref, m_hbm, g_hbm, out_ref, bm, bg, bb, sems):
    def copy_in(k, slot):
        pltpu.make_async_copy(
            m_hbm.at[pl.ds(k * _R, _R), :], bm.at[slot], sems.at[slot, 0]
        ).start()
        pltpu.make_async_copy(
            g_hbm.at[pl.ds(k * _R, _R), :], bg.at[slot], sems.at[slot, 1]
        ).start()

    def wait_in(k, slot):
        pltpu.make_async_copy(
            m_hbm.at[pl.ds(k * _R, _R), :], bm.at[slot], sems.at[slot, 0]
        ).wait()
        pltpu.make_async_copy(
            g_hbm.at[pl.ds(k * _R, _R), :], bg.at[slot], sems.at[slot, 1]
        ).wait()

    copy_in(0, 0)

    cols = jax.lax.broadcasted_iota(jnp.int32, (_ICH, _W), 1)
    zero_acc = jnp.zeros((8, 128), jnp.float32)

    xs_all = xs_ref[...]
    ys_all = ys_ref[...]
    x_min_a = jnp.maximum(
        jnp.min(xs_all, axis=1, keepdims=True).astype(jnp.int32) - 10, 0)
    x_max_a = jnp.minimum(
        jnp.max(xs_all, axis=1, keepdims=True).astype(jnp.int32) + 10, _W)
    y_min_a = jnp.maximum(
        jnp.min(ys_all, axis=1, keepdims=True).astype(jnp.int32) - 10, 0)
    y_max_a = jnp.minimum(
        jnp.max(ys_all, axis=1, keepdims=True).astype(jnp.int32) + 10, _H)
    x_len_a = jnp.maximum(x_max_a - x_min_a, 0)
    y_len_a = jnp.maximum(y_max_a - y_min_a, 0)
    area = jnp.sum((y_len_a * x_len_a).astype(jnp.float32))
    bb[0] = y_min_a
    bb[1] = y_len_a
    bb[2] = x_min_a
    bb[3] = x_len_a

    def step(k, carry):
        a_pos, a_cnt, a_all, a_ins = carry
        slot = jax.lax.rem(k, 2)

        @pl.when(k + 1 < _NCH)
        def _prefetch():
            copy_in(k + 1, jax.lax.rem(k + 1, 2))

        wait_in(k, slot)

        def chunk(c, acc3):
            c_pos, c_cnt, c_all = acc3
            x = bm[slot, pl.ds(c * _CH, _CH), :]
            g = bg[slot, pl.ds(c * _CH, _CH), :]
            y = jnp.exp2(jnp.abs(x) * jnp.float32(_NEG_LOG2E))
            l_term = jnp.log(1.0 + y)
            n = jnp.maximum(x, 0.0) + l_term
            p = n - x
            pos = g > 0.0
            c_pos = c_pos + _fold(jnp.where(pos, p, 0.0))
            c_cnt = c_cnt + _fold(jnp.where(pos, 1.0, 0.0))
            c_all = c_all + _fold(n)
            return c_pos, c_cnt, c_all

        a_pos, a_cnt, a_all = jax.lax.fori_loop(
            0, _R // _CH, chunk, (a_pos, a_cnt, a_all), unroll=8)

        for i in range(_IMGS):
            b = k * _IMGS + i
            y_min = jnp.min(bb[0, pl.ds(b, 1), :])
            y_len = jnp.min(bb[1, pl.ds(b, 1), :])
            x_min_v = bb[2, pl.ds(b, 1), :]
            x_len_v = bb[3, pl.ds(b, 1), :]

            col_in = (cols - x_min_v).astype(jnp.uint32) < x_len_v.astype(
                jnp.uint32)
            row0 = i * _H
            base = row0 + y_min
            lo = row0 // _ICH + y_min // _ICH
            hi = jnp.where(
                y_len > 0,
                row0 // _ICH + (y_min + y_len + _ICH - 1) // _ICH, lo)

            def ins_chunk(j, a, base=base, y_len=y_len, col_in=col_in,
                          slot=slot):
                xx = bm[slot, pl.ds(j * _ICH, _ICH), :]
                yy = jnp.exp2(jnp.abs(xx) * jnp.float32(_NEG_LOG2E))
                neg_val = jnp.maximum(xx, 0.0) + jnp.log(1.0 + yy)
                rows = j * _ICH + jax.lax.broadcasted_iota(
                    jnp.int32, (_ICH, _W), 0)
                row_in = (rows - base).astype(jnp.uint32) < y_len.astype(
                    jnp.uint32)
                return a + _fold_lanes(
                    jnp.where(row_in & col_in, neg_val, 0.0))

            a_ins = jax.lax.fori_loop(lo, hi, ins_chunk, a_ins)
        return a_pos, a_cnt, a_all, a_ins

    a_pos, a_cnt, a_all, a_ins = jax.lax.fori_loop(
        0, _NCH, step, (zero_acc, zero_acc, zero_acc, zero_acc))

    out_ref[0] = jnp.sum(a_pos)
    out_ref[1] = jnp.sum(a_cnt)
    out_ref[2] = jnp.sum(a_all) - jnp.sum(a_ins)
    out_ref[3] = area


def kernel(skls, masks, gt_masks):
    s = jax.lax.stop_gradient(skls)
    xs = s[:, :, 0]
    ys = s[:, :, 1]
    m2d = masks.reshape(_B * _H, _W)
    g2d = gt_masks.reshape(_B * _H, _W)

    acc = pl.pallas_call(
        _body,
        in_specs=[
            pl.BlockSpec(memory_space=pltpu.VMEM),
            pl.BlockSpec(memory_space=pltpu.VMEM),
            pl.BlockSpec(memory_space=pl.ANY),
            pl.BlockSpec(memory_space=pl.ANY),
        ],
        out_specs=pl.BlockSpec(memory_space=pltpu.SMEM),
        out_shape=jax.ShapeDtypeStruct((4,), jnp.float32),
        scratch_shapes=[
            pltpu.VMEM((2, _R, _W), jnp.float32),
            pltpu.VMEM((2, _R, _W), jnp.float32),
            pltpu.VMEM((4, _B, 1), jnp.int32),
            pltpu.SemaphoreType.DMA((2, 2)),
        ],
    )(xs, ys, m2d, g2d)

    pos_loss = acc[0] / jnp.maximum(acc[1], 1.0)
    neg_count = float(_B * _H * _W) - acc[3]
    neg_loss = acc[2] / jnp.maximum(neg_count, 1.0)
    return pos_loss + neg_loss


# speedup vs baseline: 1.0797x; 1.0074x over previous
import jax
import jax.numpy as jnp
from jax.experimental import pallas as pl
from jax.experimental.pallas import tpu as pltpu

_B, _H, _W, _J = 32, 512, 512, 17
_IMGS = 4
_R = _IMGS * _H
_NCH = _B // _IMGS
_CH = 32
_ICH = 8
_NEG_LOG2E = -1.4426950408889634


def _fold_lanes(t):
    return (t[:, 0:128] + t[:, 128:256]) + (t[:, 256:384] + t[:, 384:512])


def _fold(t):
    return _fold_lanes((t[0:8] + t[8:16]) + (t[16:24] + t[24:32]))


def _body(xs_ref, ys_ref, m_hbm, g_hbm, out_ref, bm, bg, bb, sems):
    def copy_in(k, slot):
        pltpu.make_async_copy(
            m_hbm.at[pl.ds(k * _R, _R), :], bm.at[slot], sems.at[slot, 0]
        ).start()
        pltpu.make_async_copy(
            g_hbm.at[pl.ds(k * _R, _R), :], bg.at[slot], sems.at[slot, 1]
        ).start()

    def wait_in(k, slot):
        pltpu.make_async_copy(
            m_hbm.at[pl.ds(k * _R, _R), :], bm.at[slot], sems.at[slot, 0]
        ).wait()
        pltpu.make_async_copy(
            g_hbm.at[pl.ds(k * _R, _R), :], bg.at[slot], sems.at[slot, 1]
        ).wait()

    copy_in(0, 0)

    cols = jax.lax.broadcasted_iota(jnp.int32, (_ICH, _W), 1)
    zero_acc = jnp.zeros((8, 128), jnp.float32)

    xs_all = xs_ref[...]
    ys_all = ys_ref[...]
    x_min_a = jnp.maximum(
        jnp.min(xs_all, axis=1, keepdims=True).astype(jnp.int32) - 10, 0)
    x_max_a = jnp.minimum(
        jnp.max(xs_all, axis=1, keepdims=True).astype(jnp.int32) + 10, _W)
    y_min_a = jnp.maximum(
        jnp.min(ys_all, axis=1, keepdims=True).astype(jnp.int32) - 10, 0)
    y_max_a = jnp.minimum(
        jnp.max(ys_all, axis=1, keepdims=True).astype(jnp.int32) + 10, _H)
    x_len_a = jnp.maximum(x_max_a - x_min_a, 0)
    y_len_a = jnp.maximum(y_max_a - y_min_a, 0)
    area = jnp.sum((y_len_a * x_len_a).astype(jnp.float32))
    bb[0] = y_min_a
    bb[1] = y_len_a
    bb[2] = x_min_a
    bb[3] = x_len_a

    def step(k, carry):
        a_pos, a_cnt, a_all, a_ins = carry
        slot = jax.lax.rem(k, 2)

        @pl.when(k + 1 < _NCH)
        def _prefetch():
            copy_in(k + 1, jax.lax.rem(k + 1, 2))

        wait_in(k, slot)

        def chunk(c, acc3):
            c_pos, c_cnt, c_all = acc3
            x = bm[slot, pl.ds(c * _CH, _CH), :]
            g = bg[slot, pl.ds(c * _CH, _CH), :]
            y = jnp.exp2(jnp.abs(x) * jnp.float32(_NEG_LOG2E))
            l_term = jnp.log(1.0 + y)
            n = jnp.maximum(x, 0.0) + l_term
            p = n - x
            pos = g > 0.0
            c_pos = c_pos + _fold(jnp.where(pos, p, 0.0))
            c_cnt = c_cnt + _fold(jnp.where(pos, 1.0, 0.0))
            c_all = c_all + _fold(n)
            return c_pos, c_cnt, c_all

        a_pos, a_cnt, a_all = jax.lax.fori_loop(
            0, _R // _CH, chunk, (a_pos, a_cnt, a_all), unroll=16)

        for i in range(_IMGS):
            b = k * _IMGS + i
            y_min = jnp.min(bb[0, pl.ds(b, 1), :])
            y_len = jnp.min(bb[1, pl.ds(b, 1), :])
            x_min_v = bb[2, pl.ds(b, 1), :]
            x_len_v = bb[3, pl.ds(b, 1), :]

            col_in = (cols - x_min_v).astype(jnp.uint32) < x_len_v.astype(
                jnp.uint32)
            row0 = i * _H
            base = row0 + y_min
            lo = row0 // _ICH + y_min // _ICH
            hi = jnp.where(
                y_len > 0,
                row0 // _ICH + (y_min + y_len + _ICH - 1) // _ICH, lo)

            def ins_chunk(j, a, base=base, y_len=y_len, col_in=col_in,
                          slot=slot):
                xx = bm[slot, pl.ds(j * _ICH, _ICH), :]
                yy = jnp.exp2(jnp.abs(xx) * jnp.float32(_NEG_LOG2E))
                neg_val = jnp.maximum(xx, 0.0) + jnp.log(1.0 + yy)
                rows = j * _ICH + jax.lax.broadcasted_iota(
                    jnp.int32, (_ICH, _W), 0)
                row_in = (rows - base).astype(jnp.uint32) < y_len.astype(
                    jnp.uint32)
                return a + _fold_lanes(
                    jnp.where(row_in & col_in, neg_val, 0.0))

            a_ins = jax.lax.fori_loop(lo, hi, ins_chunk, a_ins)
        return a_pos, a_cnt, a_all, a_ins

    a_pos, a_cnt, a_all, a_ins = jax.lax.fori_loop(
        0, _NCH, step, (zero_acc, zero_acc, zero_acc, zero_acc))

    out_ref[0] = jnp.sum(a_pos)
    out_ref[1] = jnp.sum(a_cnt)
    out_ref[2] = jnp.sum(a_all) - jnp.sum(a_ins)
    out_ref[3] = area


def kernel(skls, masks, gt_masks):
    s = jax.lax.stop_gradient(skls)
    xs = s[:, :, 0]
    ys = s[:, :, 1]
    m2d = masks.reshape(_B * _H, _W)
    g2d = gt_masks.reshape(_B * _H, _W)

    acc = pl.pallas_call(
        _body,
        in_specs=[
            pl.BlockSpec(memory_space=pltpu.VMEM),
            pl.BlockSpec(memory_space=pltpu.VMEM),
            pl.BlockSpec(memory_space=pl.ANY),
            pl.BlockSpec(memory_space=pl.ANY),
        ],
        out_specs=pl.BlockSpec(memory_space=pltpu.SMEM),
        out_shape=jax.ShapeDtypeStruct((4,), jnp.float32),
        scratch_shapes=[
            pltpu.VMEM((2, _R, _W), jnp.float32),
            pltpu.VMEM((2, _R, _W), jnp.float32),
            pltpu.VMEM((4, _B, 1), jnp.int32),
            pltpu.SemaphoreType.DMA((2, 2)),
        ],
    )(xs, ys, m2d, g2d)

    pos_loss = acc[0] / jnp.maximum(acc[1], 1.0)
    neg_count = float(_B * _H * _W) - acc[3]
    neg_loss = acc[2] / jnp.maximum(neg_count, 1.0)
    return pos_loss + neg_loss


# speedup vs baseline: 1.0866x; 1.0063x over previous
import jax
import jax.numpy as jnp
from jax.experimental import pallas as pl
from jax.experimental.pallas import tpu as pltpu

_B, _H, _W, _J = 32, 512, 512, 17
_IMGS = 4
_R = _IMGS * _H
_NCH = _B // _IMGS
_CH = 32
_ICH = 8
_NEG_LOG2E = -1.4426950408889634


def _fold_lanes(t):
    return (t[:, 0:128] + t[:, 128:256]) + (t[:, 256:384] + t[:, 384:512])


def _fold(t):
    return _fold_lanes((t[0:8] + t[8:16]) + (t[16:24] + t[24:32]))


def _body(xs_ref, ys_ref, m_hbm, g_hbm, out_ref, bm, bg, bb, sems):
    def copy_in(k, slot):
        pltpu.make_async_copy(
            m_hbm.at[pl.ds(k * _R, _R), :], bm.at[slot], sems.at[slot, 0]
        ).start()
        pltpu.make_async_copy(
            g_hbm.at[pl.ds(k * _R, _R), :], bg.at[slot], sems.at[slot, 1]
        ).start()

    def wait_in(k, slot):
        pltpu.make_async_copy(
            m_hbm.at[pl.ds(k * _R, _R), :], bm.at[slot], sems.at[slot, 0]
        ).wait()
        pltpu.make_async_copy(
            g_hbm.at[pl.ds(k * _R, _R), :], bg.at[slot], sems.at[slot, 1]
        ).wait()

    copy_in(0, 0)

    cols = jax.lax.broadcasted_iota(jnp.int32, (_ICH, _W), 1)
    zero_acc = jnp.zeros((8, 128), jnp.float32)

    xs_all = xs_ref[...]
    ys_all = ys_ref[...]
    x_min_a = jnp.maximum(
        jnp.min(xs_all, axis=1, keepdims=True).astype(jnp.int32) - 10, 0)
    x_max_a = jnp.minimum(
        jnp.max(xs_all, axis=1, keepdims=True).astype(jnp.int32) + 10, _W)
    y_min_a = jnp.maximum(
        jnp.min(ys_all, axis=1, keepdims=True).astype(jnp.int32) - 10, 0)
    y_max_a = jnp.minimum(
        jnp.max(ys_all, axis=1, keepdims=True).astype(jnp.int32) + 10, _H)
    x_len_a = jnp.maximum(x_max_a - x_min_a, 0)
    y_len_a = jnp.maximum(y_max_a - y_min_a, 0)
    area = jnp.sum((y_len_a * x_len_a).astype(jnp.float32))
    bb[0] = y_min_a
    bb[1] = y_len_a
    bb[2] = x_min_a
    bb[3] = x_len_a

    def step(k, carry):
        a_pos, a_cnt, a_all, a_ins = carry
        slot = jax.lax.rem(k, 2)

        @pl.when(k + 1 < _NCH)
        def _prefetch():
            copy_in(k + 1, jax.lax.rem(k + 1, 2))

        wait_in(k, slot)

        def chunk(c, acc3):
            c_pos, c_cnt, c_all = acc3
            x = bm[slot, pl.ds(c * _CH, _CH), :]
            g = bg[slot, pl.ds(c * _CH, _CH), :]
            y = jnp.exp2(jnp.abs(x) * jnp.float32(_NEG_LOG2E))
            l_term = jnp.log(1.0 + y)
            n = jnp.maximum(x, 0.0) + l_term
            p = n - x
            pos = g > 0.0
            c_pos = c_pos + _fold(jnp.where(pos, p, 0.0))
            c_cnt = c_cnt + _fold(jnp.where(pos, 1.0, 0.0))
            c_all = c_all + _fold(n)
            return c_pos, c_cnt, c_all

        a_pos, a_cnt, a_all = jax.lax.fori_loop(
            0, _R // _CH, chunk, (a_pos, a_cnt, a_all), unroll=32)

        for i in range(_IMGS):
            b = k * _IMGS + i
            y_min = jnp.min(bb[0, pl.ds(b, 1), :])
            y_len = jnp.min(bb[1, pl.ds(b, 1), :])
            x_min_v = bb[2, pl.ds(b, 1), :]
            x_len_v = bb[3, pl.ds(b, 1), :]

            col_in = (cols - x_min_v).astype(jnp.uint32) < x_len_v.astype(
                jnp.uint32)
            row0 = i * _H
            base = row0 + y_min
            lo = row0 // _ICH + y_min // _ICH
            hi = jnp.where(
                y_len > 0,
                row0 // _ICH + (y_min + y_len + _ICH - 1) // _ICH, lo)

            def ins_chunk(j, a, base=base, y_len=y_len, col_in=col_in,
                          slot=slot):
                xx = bm[slot, pl.ds(j * _ICH, _ICH), :]
                yy = jnp.exp2(jnp.abs(xx) * jnp.float32(_NEG_LOG2E))
                neg_val = jnp.maximum(xx, 0.0) + jnp.log(1.0 + yy)
                rows = j * _ICH + jax.lax.broadcasted_iota(
                    jnp.int32, (_ICH, _W), 0)
                row_in = (rows - base).astype(jnp.uint32) < y_len.astype(
                    jnp.uint32)
                return a + _fold_lanes(
                    jnp.where(row_in & col_in, neg_val, 0.0))

            a_ins = jax.lax.fori_loop(lo, hi, ins_chunk, a_ins)
        return a_pos, a_cnt, a_all, a_ins

    a_pos, a_cnt, a_all, a_ins = jax.lax.fori_loop(
        0, _NCH, step, (zero_acc, zero_acc, zero_acc, zero_acc))

    out_ref[0] = jnp.sum(a_pos)
    out_ref[1] = jnp.sum(a_cnt)
    out_ref[2] = jnp.sum(a_all) - jnp.sum(a_ins)
    out_ref[3] = area


def kernel(skls, masks, gt_masks):
    s = jax.lax.stop_gradient(skls)
    xs = s[:, :, 0]
    ys = s[:, :, 1]
    m2d = masks.reshape(_B * _H, _W)
    g2d = gt_masks.reshape(_B * _H, _W)

    acc = pl.pallas_call(
        _body,
        in_specs=[
            pl.BlockSpec(memory_space=pltpu.VMEM),
            pl.BlockSpec(memory_space=pltpu.VMEM),
            pl.BlockSpec(memory_space=pl.ANY),
            pl.BlockSpec(memory_space=pl.ANY),
        ],
        out_specs=pl.BlockSpec(memory_space=pltpu.SMEM),
        out_shape=jax.ShapeDtypeStruct((4,), jnp.float32),
        scratch_shapes=[
            pltpu.VMEM((2, _R, _W), jnp.float32),
            pltpu.VMEM((2, _R, _W), jnp.float32),
            pltpu.VMEM((4, _B, 1), jnp.int32),
            pltpu.SemaphoreType.DMA((2, 2)),
        ],
    )(xs, ys, m2d, g2d)

    pos_loss = acc[0] / jnp.maximum(acc[1], 1.0)
    neg_count = float(_B * _H * _W) - acc[3]
    neg_loss = acc[2] / jnp.maximum(neg_count, 1.0)
    return pos_loss + neg_loss
